# baseline (speedup 1.0000x reference)
import functools
import math

import jax
import jax.numpy as jnp
from jax import lax
from jax.experimental import pallas as pl
from jax.experimental.pallas import tpu as pltpu
from jax.experimental.pallas import tpu_sc as plsc

D_MODEL = 64
SCALE = math.sqrt(D_MODEL)

SEQ = 200
BATCH = 4096
VOCAB = 1000000
N_TOKENS = BATCH * SEQ
NUM_WORKERS = 32

B = 128
BLK_PER_J = BATCH // B
NBLK = N_TOKENS // B
PER_W = NBLK // NUM_WORKERS
NS = 4
GROUPS = PER_W // NS


V_BLK = 8192
N_VBLK = -(-VOCAB // V_BLK)


def _pad_body(wt_ref, out_ref):
    t = jnp.transpose(wt_ref[...]) * SCALE
    out_ref[...] = jnp.concatenate(
        [t, jnp.zeros((V_BLK, 128 - D_MODEL), jnp.float32)], axis=1
    )


_pad_table = pl.pallas_call(
    _pad_body,
    grid=(N_VBLK,),
    in_specs=[pl.BlockSpec((D_MODEL, V_BLK), lambda i: (0, i))],
    out_specs=pl.BlockSpec((V_BLK, 128), lambda i: (i, 0)),
    out_shape=jax.ShapeDtypeStruct((VOCAB, 128), jnp.float32),
)


_MESH = plsc.VectorSubcoreMesh(core_axis_name="c", subcore_axis_name="s")


@functools.partial(
    pl.kernel,
    out_type=jax.ShapeDtypeStruct((SEQ, D_MODEL, BATCH), jnp.float32),
    mesh=_MESH,
    scratch_types=[
        [pltpu.VMEM((B,), jnp.int32) for _ in range(NS)],
        [pltpu.VMEM((B, 128), jnp.float32) for _ in range(NS)],
        [pltpu.VMEM((D_MODEL, B), jnp.float32) for _ in range(NS)],
        [pltpu.SemaphoreType.DMA for _ in range(NS)],
        [pltpu.SemaphoreType.DMA for _ in range(NS)],
        [pltpu.SemaphoreType.DMA for _ in range(NS)],
    ],
    compiler_params=pltpu.CompilerParams(needs_layout_passes=False),
)
def _gather(x_hbm, w_hbm, out_hbm, idxs, gs, ts, sem_i, sem_g, sem_st):
    wid = lax.axis_index("s") * 2 + lax.axis_index("c")
    bid0 = wid * PER_W
    bid_end = bid0 + PER_W

    def block_coords(bid):
        return bid // BLK_PER_J, (bid % BLK_PER_J) * B

    def issue_idx(bid, s):
        j, i0 = block_coords(bid)
        pltpu.async_copy(x_hbm.at[j, pl.ds(i0, B)], idxs[s], sem_i[s])

    def wait_idx(s):
        pltpu.make_async_copy(
            x_hbm.at[0, pl.ds(0, B)], idxs[s], sem_i[s]
        ).wait()

    def issue_gather(s):
        pltpu.async_copy(w_hbm.at[idxs[s]], gs[s], sem_g[s])

    def wait_gather(s):
        pltpu.make_async_copy(w_hbm.at[idxs[s]], gs[s], sem_g[s]).wait()

    def issue_store(bid, s):
        j, i0 = block_coords(bid)
        pltpu.async_copy(ts[s], out_hbm.at[j, :, pl.ds(i0, B)], sem_st[s])

    def wait_store(s):
        pltpu.make_async_copy(
            ts[s], out_hbm.at[0, :, pl.ds(0, B)], sem_st[s]
        ).wait()

    lanes = lax.iota(jnp.int32, 16)
    diag_rows = [(lanes + k) & 15 for k in range(16)]
    d_cols = [lanes + 16 * d16 for d16 in range(D_MODEL // 16)]

    def transform(s):
        src = gs[s]
        dst = ts[s]

        @plsc.parallel_loop(0, B // 16, 1, unroll=2)
        def _(ii):
            i0 = ii * 16
            srow = src.at[pl.ds(i0, 16), :]
            for k in range(16):
                r = diag_rows[k] + i0
                for d16 in range(D_MODEL // 16):
                    v = plsc.load_gather(srow, [diag_rows[k], d_cols[d16]])
                    plsc.store_scatter(dst, [d_cols[d16], r], v)

    for s in range(NS):
        issue_idx(bid0 + s, s)
    for s in range(2):
        wait_idx(s)
        issue_gather(s)

    def group_body(g, carry):
        for b in range(NS):
            bid = bid0 + g * NS + b
            s_ahead = (b + 2) % NS

            @pl.when(bid + 2 < bid_end)
            def _():
                wait_idx(s_ahead)
                issue_gather(s_ahead)

            wait_gather(b)

            @pl.when(g > 0)
            def _():
                wait_store(b)

            transform(b)
            issue_store(bid, b)

            @pl.when(bid + NS < bid_end)
            def _():
                issue_idx(bid + NS, b)

        return carry

    lax.fori_loop(0, GROUPS, group_body, 0)

    for b in range(NS):
        wait_store(b)


def kernel(x, weight):
    w_pad = _pad_table(weight.T)
    out_t = _gather(x.T, w_pad)
    return jnp.transpose(out_t, (2, 0, 1))


# speedup vs baseline: 1.0661x; 1.0661x over previous
import functools
import math

import jax
import jax.numpy as jnp
from jax import lax
from jax.experimental import pallas as pl
from jax.experimental.pallas import tpu as pltpu
from jax.experimental.pallas import tpu_sc as plsc

D_MODEL = 64
SCALE = math.sqrt(D_MODEL)

SEQ = 200
BATCH = 4096
VOCAB = 1000000
N_TOKENS = BATCH * SEQ
NUM_WORKERS = 32

B = 256
BLK_PER_J = BATCH // B
NBLK = N_TOKENS // B
PER_W = NBLK // NUM_WORKERS
NS = 2
GROUPS = PER_W // NS


V_BLK = 8192
N_VBLK = -(-VOCAB // V_BLK)


def _pad_body(wt_ref, out_ref):
    t = jnp.transpose(wt_ref[...]) * SCALE
    out_ref[...] = jnp.concatenate(
        [t, jnp.zeros((V_BLK, 128 - D_MODEL), jnp.float32)], axis=1
    )


_pad_table = pl.pallas_call(
    _pad_body,
    grid=(N_VBLK,),
    in_specs=[pl.BlockSpec((D_MODEL, V_BLK), lambda i: (0, i))],
    out_specs=pl.BlockSpec((V_BLK, 128), lambda i: (i, 0)),
    out_shape=jax.ShapeDtypeStruct((VOCAB, 128), jnp.float32),
)


_MESH = plsc.VectorSubcoreMesh(core_axis_name="c", subcore_axis_name="s")


@functools.partial(
    pl.kernel,
    out_type=jax.ShapeDtypeStruct((SEQ, D_MODEL, BATCH), jnp.float32),
    mesh=_MESH,
    scratch_types=[
        [pltpu.VMEM((B,), jnp.int32) for _ in range(NS)],
        [pltpu.VMEM((B, 128), jnp.float32) for _ in range(NS)],
        [pltpu.VMEM((D_MODEL, B), jnp.float32) for _ in range(NS)],
        [pltpu.SemaphoreType.DMA for _ in range(NS)],
        [pltpu.SemaphoreType.DMA for _ in range(NS)],
        [pltpu.SemaphoreType.DMA for _ in range(NS)],
    ],
    compiler_params=pltpu.CompilerParams(needs_layout_passes=False),
)
def _gather(x_hbm, w_hbm, out_hbm, idxs, gs, ts, sem_i, sem_g, sem_st):
    wid = lax.axis_index("s") * 2 + lax.axis_index("c")
    bid0 = wid * PER_W
    bid_end = bid0 + PER_W

    def block_coords(bid):
        return bid // BLK_PER_J, (bid % BLK_PER_J) * B

    def issue_idx(bid, s):
        j, i0 = block_coords(bid)
        pltpu.async_copy(x_hbm.at[j, pl.ds(i0, B)], idxs[s], sem_i[s])

    def wait_idx(s):
        pltpu.make_async_copy(
            x_hbm.at[0, pl.ds(0, B)], idxs[s], sem_i[s]
        ).wait()

    def issue_gather(s):
        pltpu.async_copy(w_hbm.at[idxs[s]], gs[s], sem_g[s])

    def wait_gather(s):
        pltpu.make_async_copy(w_hbm.at[idxs[s]], gs[s], sem_g[s]).wait()

    def issue_store(bid, s):
        j, i0 = block_coords(bid)
        pltpu.async_copy(ts[s], out_hbm.at[j, :, pl.ds(i0, B)], sem_st[s])

    def wait_store(s):
        pltpu.make_async_copy(
            ts[s], out_hbm.at[0, :, pl.ds(0, B)], sem_st[s]
        ).wait()

    lanes = lax.iota(jnp.int32, 16)
    diag_rows = [(lanes + k) & 15 for k in range(16)]
    d_cols = [lanes + 16 * d16 for d16 in range(D_MODEL // 16)]

    def transform(s):
        src = gs[s]
        dst = ts[s]

        @plsc.parallel_loop(0, B // 16, 1)
        def _(ii):
            i0 = ii * 16
            srow = src.at[pl.ds(i0, 16), :]
            for k in range(16):
                r = diag_rows[k] + i0
                for d16 in range(D_MODEL // 16):
                    v = plsc.load_gather(srow, [diag_rows[k], d_cols[d16]])
                    plsc.store_scatter(dst, [d_cols[d16], r], v)

    for s in range(NS):
        issue_idx(bid0 + s, s)
    for s in range(1):
        wait_idx(s)
        issue_gather(s)

    def group_body(g, carry):
        for b in range(NS):
            bid = bid0 + g * NS + b
            s_ahead = (b + 1) % NS

            @pl.when(bid + 1 < bid_end)
            def _():
                wait_idx(s_ahead)
                issue_gather(s_ahead)

            wait_gather(b)

            @pl.when(g > 0)
            def _():
                wait_store(b)

            transform(b)
            issue_store(bid, b)

            @pl.when(bid + NS < bid_end)
            def _():
                issue_idx(bid + NS, b)

        return carry

    lax.fori_loop(0, GROUPS, group_body, 0)

    for b in range(NS):
        wait_store(b)


def kernel(x, weight):
    w_pad = _pad_table(weight.T)
    out_t = _gather(x.T, w_pad)
    return jnp.transpose(out_t, (2, 0, 1))


# speedup vs baseline: 1.1622x; 1.0902x over previous
import functools
import math

import jax
import jax.numpy as jnp
from jax import lax
from jax.experimental import pallas as pl
from jax.experimental.pallas import tpu as pltpu
from jax.experimental.pallas import tpu_sc as plsc

D_MODEL = 64
SCALE = math.sqrt(D_MODEL)

SEQ = 200
BATCH = 4096
VOCAB = 1000000
N_TOKENS = BATCH * SEQ
NUM_WORKERS = 32

B = 128
BLK_PER_J = BATCH // B
NBLK = N_TOKENS // B
PER_W = NBLK // NUM_WORKERS
NS = 4
GROUPS = PER_W // NS


V_BLK = 8192
N_VBLK = -(-VOCAB // V_BLK)


def _pad_body(wt_ref, out_ref):
    t = jnp.transpose(wt_ref[...]) * SCALE
    out_ref[...] = jnp.concatenate(
        [t, jnp.zeros((V_BLK, 128 - D_MODEL), jnp.float32)], axis=1
    )


_pad_table = pl.pallas_call(
    _pad_body,
    grid=(N_VBLK,),
    in_specs=[pl.BlockSpec((D_MODEL, V_BLK), lambda i: (0, i))],
    out_specs=pl.BlockSpec((V_BLK, 128), lambda i: (i, 0)),
    out_shape=jax.ShapeDtypeStruct((VOCAB, 128), jnp.float32),
)


_MESH = plsc.VectorSubcoreMesh(core_axis_name="c", subcore_axis_name="s")


@functools.partial(
    pl.kernel,
    out_type=jax.ShapeDtypeStruct((SEQ, D_MODEL, BATCH), jnp.float32),
    mesh=_MESH,
    scratch_types=[
        [pltpu.VMEM((B,), jnp.int32) for _ in range(NS)],
        [pltpu.VMEM((B, 128), jnp.float32) for _ in range(NS)],
        [pltpu.VMEM((D_MODEL, B), jnp.float32) for _ in range(NS)],
        [pltpu.SemaphoreType.DMA for _ in range(NS)],
        [pltpu.SemaphoreType.DMA for _ in range(NS)],
        [pltpu.SemaphoreType.DMA for _ in range(NS)],
    ],
    compiler_params=pltpu.CompilerParams(needs_layout_passes=False),
)
def _gather(x_hbm, w_hbm, out_hbm, idxs, gs, ts, sem_i, sem_g, sem_st):
    wid = lax.axis_index("s") * 2 + lax.axis_index("c")
    bid0 = wid * PER_W
    bid_end = bid0 + PER_W

    def block_coords(bid):
        return bid // BLK_PER_J, (bid % BLK_PER_J) * B

    def issue_idx(bid, s):
        j, i0 = block_coords(bid)
        pltpu.async_copy(x_hbm.at[j, pl.ds(i0, B)], idxs[s], sem_i[s])

    def wait_idx(s):
        pltpu.make_async_copy(
            x_hbm.at[0, pl.ds(0, B)], idxs[s], sem_i[s]
        ).wait()

    def issue_gather(s):
        pltpu.async_copy(w_hbm.at[idxs[s]], gs[s], sem_g[s])

    def wait_gather(s):
        pltpu.make_async_copy(w_hbm.at[idxs[s]], gs[s], sem_g[s]).wait()

    def issue_store(bid, s):
        j, i0 = block_coords(bid)
        pltpu.async_copy(ts[s], out_hbm.at[j, :, pl.ds(i0, B)], sem_st[s])

    def wait_store(s):
        pltpu.make_async_copy(
            ts[s], out_hbm.at[0, :, pl.ds(0, B)], sem_st[s]
        ).wait()

    lanes = lax.iota(jnp.int32, 16)
    diag_rows = [(lanes + k) & 15 for k in range(16)]
    d_cols = [lanes + 16 * d16 for d16 in range(D_MODEL // 16)]

    def transform(s):
        src = gs[s]
        dst = ts[s]

        @plsc.parallel_loop(0, B // 16, 1)
        def _(ii):
            i0 = ii * 16
            srow = src.at[pl.ds(i0, 16), :]
            for k in range(16):
                r = diag_rows[k] + i0
                for d16 in range(D_MODEL // 16):
                    v = plsc.load_gather(srow, [diag_rows[k], d_cols[d16]])
                    plsc.store_scatter(dst, [d_cols[d16], r], v)

    for s in range(NS):
        issue_idx(bid0 + s, s)
    for s in range(2):
        wait_idx(s)
        issue_gather(s)

    def group_body(g, carry):
        for b in range(NS):
            bid = bid0 + g * NS + b
            s_ahead = (b + 2) % NS

            @pl.when(bid + 2 < bid_end)
            def _():
                wait_idx(s_ahead)
                issue_gather(s_ahead)

            wait_gather(b)

            @pl.when(g > 0)
            def _():
                wait_store(b)

            transform(b)
            issue_store(bid, b)

            @pl.when(bid + NS < bid_end)
            def _():
                issue_idx(bid + NS, b)

        return carry

    lax.fori_loop(0, GROUPS, group_body, 0)

    for b in range(NS):
        wait_store(b)


def kernel(x, weight):
    w_pad = _pad_table(weight.T)
    out_t = _gather(x.T, w_pad)
    return jnp.transpose(out_t, (2, 0, 1))


# speedup vs baseline: 1.2142x; 1.0447x over previous
import functools
import math

import jax
import jax.numpy as jnp
from jax import lax
from jax.experimental import pallas as pl
from jax.experimental.pallas import tpu as pltpu
from jax.experimental.pallas import tpu_sc as plsc

D_MODEL = 64
SCALE = math.sqrt(D_MODEL)

SEQ = 200
BATCH = 4096
VOCAB = 1000000
N_TOKENS = BATCH * SEQ
NUM_WORKERS = 32

B = 128
BLK_PER_J = BATCH // B
NBLK = N_TOKENS // B
PER_W = NBLK // NUM_WORKERS
NS = 4
GROUPS = PER_W // (2 * NS)


V_BLK = 8192
N_VBLK = -(-VOCAB // V_BLK)


def _pad_body(wt_ref, out_ref):
    t = jnp.transpose(wt_ref[...]) * SCALE
    out_ref[...] = jnp.concatenate(
        [t, jnp.zeros((V_BLK, 128 - D_MODEL), jnp.float32)], axis=1
    )


_pad_table = pl.pallas_call(
    _pad_body,
    grid=(N_VBLK,),
    in_specs=[pl.BlockSpec((D_MODEL, V_BLK), lambda i: (0, i))],
    out_specs=pl.BlockSpec((V_BLK, 128), lambda i: (i, 0)),
    out_shape=jax.ShapeDtypeStruct((VOCAB, 128), jnp.float32),
)


_MESH = plsc.VectorSubcoreMesh(core_axis_name="c", subcore_axis_name="s")


@functools.partial(
    pl.kernel,
    out_type=jax.ShapeDtypeStruct((SEQ, D_MODEL, BATCH), jnp.float32),
    mesh=_MESH,
    scratch_types=[
        [pltpu.VMEM((B,), jnp.int32) for _ in range(2 * NS)],
        [pltpu.VMEM((B, 128), jnp.float32) for _ in range(NS)],
        [pltpu.VMEM((D_MODEL, B), jnp.float32) for _ in range(NS)],
        [pltpu.SemaphoreType.DMA for _ in range(2 * NS)],
        [pltpu.SemaphoreType.DMA for _ in range(NS)],
        [pltpu.SemaphoreType.DMA for _ in range(NS)],
    ],
    compiler_params=pltpu.CompilerParams(needs_layout_passes=False),
)
def _gather(x_hbm, w_hbm, out_hbm, idxs, gs, ts, sem_i, sem_g, sem_st):
    wid = lax.axis_index("s") * 2 + lax.axis_index("c")
    bid0 = wid * PER_W
    bid_end = bid0 + PER_W

    def block_coords(bid):
        return bid // BLK_PER_J, (bid % BLK_PER_J) * B

    def issue_idx(bid, s):
        j, i0 = block_coords(bid)
        pltpu.async_copy(x_hbm.at[j, pl.ds(i0, B)], idxs[s], sem_i[s])

    def wait_idx(s):
        pltpu.make_async_copy(
            x_hbm.at[0, pl.ds(0, B)], idxs[s], sem_i[s]
        ).wait()

    def issue_gather(si, s):
        pltpu.async_copy(w_hbm.at[idxs[si]], gs[s], sem_g[s])

    def wait_gather(s):
        pltpu.make_async_copy(w_hbm.at[idxs[0]], gs[s], sem_g[s]).wait()

    def issue_store(bid, s):
        j, i0 = block_coords(bid)
        pltpu.async_copy(ts[s], out_hbm.at[j, :, pl.ds(i0, B)], sem_st[s])

    def wait_store(s):
        pltpu.make_async_copy(
            ts[s], out_hbm.at[0, :, pl.ds(0, B)], sem_st[s]
        ).wait()

    lanes = lax.iota(jnp.int32, 16)
    diag_rows = [(lanes + k) & 15 for k in range(16)]
    d_cols = [lanes + 16 * d16 for d16 in range(D_MODEL // 16)]

    def transform(s):
        src = gs[s]
        dst = ts[s]

        @plsc.parallel_loop(0, B // 16, 1)
        def _(ii):
            i0 = ii * 16
            srow = src.at[pl.ds(i0, 16), :]

            def kbody(k, c):
                rows = (lanes + k) & 15
                r = rows + i0
                for d16 in range(D_MODEL // 16):
                    v = plsc.load_gather(srow, [rows, d_cols[d16]])
                    plsc.store_scatter(dst, [d_cols[d16], r], v)
                return c

            lax.fori_loop(0, 16, kbody, 0)

    K = 3
    for s in range(2 * NS):
        issue_idx(bid0 + s, s)
    for s in range(K):
        wait_idx(s)
        issue_gather(s, s)

    def group_body(g, carry):
        for b in range(2 * NS):
            bid = bid0 + g * 2 * NS + b
            sg = b % NS
            gi_ahead = (b + K) % (2 * NS)
            sg_ahead = (b + K) % NS

            @pl.when(bid + K < bid_end)
            def _():
                wait_idx(gi_ahead)
                issue_gather(gi_ahead, sg_ahead)

            wait_gather(sg)

            @pl.when(bid + 2 * NS < bid_end)
            def _():
                issue_idx(bid + 2 * NS, b)

            if b >= NS:
                wait_store(sg)
            else:
                @pl.when(g > 0)
                def _():
                    wait_store(sg)

            transform(sg)
            issue_store(bid, sg)

        return carry

    lax.fori_loop(0, GROUPS, group_body, 0)

    for b in range(NS):
        wait_store(b)


def kernel(x, weight):
    w_pad = _pad_table(weight.T)
    out_t = _gather(x.T, w_pad)
    return jnp.transpose(out_t, (2, 0, 1))
